# baseline (speedup 1.0000x reference)
import jax
import jax.numpy as jnp
from jax import lax
from jax.experimental import pallas as pl
from jax.experimental.pallas import tpu as pltpu
from jax.experimental.pallas import tpu_sc as plsc

VOCAB = 1000000
DIM = 64
NEG = 5
BATCH = 16384

NC, NS, L = 2, 16, 16
NW = NC * NS
B_PER_W = BATCH // NW
C = 128
N_CHUNKS = B_PER_W // C
G = C // L
PAIR = 2 * DIM


def _sc_body(dataf_hbm, emb0p_hbm, emb1p_hbm, out_hbm,
             data_v, w_idx, c_idx, n_idx, w_par, c_par, n_par,
             w_rows, c_rows, n_rows, out_v, sem):
    wid = lax.axis_index("c") * NS + lax.axis_index("s")
    base = wid * B_PER_W
    iota = lax.iota(jnp.int32, L)

    def chunk_body(ch, _):
        cbase = base + ch * C
        pltpu.sync_copy(dataf_hbm.at[pl.ds(cbase * 12, C * 12)],
                        data_v.at[pl.ds(0, C * 12)])

        for g in range(G):
            sl = pl.ds(g * L, L)
            rows12 = (g * L + iota) * 12
            vw = plsc.load_gather(data_v, [rows12])
            vc = plsc.load_gather(data_v, [rows12 + 1])
            w_idx[sl] = vw >> 1
            w_par[sl] = (vw & 1) * DIM
            c_idx[sl] = vc >> 1
            c_par[sl] = (vc & 1) * DIM
            for k in range(NEG):
                vn = plsc.load_gather(data_v, [rows12 + (2 + k)])
                n_idx[k, sl] = vn >> 1
                n_par[k, sl] = (vn & 1) * DIM

        cps = [pltpu.async_copy(emb0p_hbm.at[w_idx], w_rows, sem),
               pltpu.async_copy(emb1p_hbm.at[c_idx], c_rows, sem)]
        for k in range(NEG):
            cps.append(pltpu.async_copy(emb1p_hbm.at[n_idx.at[k]],
                                        n_rows.at[k], sem))
        for cp in cps:
            cp.wait()

        def group_body(g, _):
            sl = pl.ds(g * L, L)
            rows = g * L + iota
            wcol = w_par[sl]
            ccol = c_par[sl]
            ncol = [n_par[k, sl] for k in range(NEG)]
            accp = jnp.zeros((L,), jnp.float32)
            accn = [jnp.zeros((L,), jnp.float32) for _ in range(NEG)]
            for d in range(DIM):
                wv = plsc.load_gather(w_rows, [rows, wcol + d])
                cv = plsc.load_gather(c_rows, [rows, ccol + d])
                accp = accp + wv * cv
                for k in range(NEG):
                    nv = plsc.load_gather(
                        n_rows, [jnp.full((L,), k, jnp.int32), rows,
                                 ncol[k] + d])
                    accn[k] = accn[k] + wv * nv
            obase = rows * 8
            plsc.store_scatter(out_v, [obase], accp)
            for k in range(NEG):
                plsc.store_scatter(out_v, [obase + (k + 1)], accn[k])
            return 0

        lax.fori_loop(0, G, group_body, 0)
        pltpu.sync_copy(out_v, out_hbm.at[pl.ds(cbase * 8, C * 8)])
        return 0

    lax.fori_loop(0, N_CHUNKS, chunk_body, 0)


def _sc_ips(data_flat, emb0p, emb1p):
    mesh = plsc.VectorSubcoreMesh(core_axis_name="c", subcore_axis_name="s")
    return pl.kernel(
        _sc_body,
        out_type=jax.ShapeDtypeStruct((BATCH * 8,), jnp.float32),
        mesh=mesh,
        compiler_params=pltpu.CompilerParams(needs_layout_passes=False),
        scratch_types=[
            pltpu.VMEM((C * 12 + L,), jnp.int32),
            pltpu.VMEM((C,), jnp.int32),
            pltpu.VMEM((C,), jnp.int32),
            pltpu.VMEM((NEG, C), jnp.int32),
            pltpu.VMEM((C,), jnp.int32),
            pltpu.VMEM((C,), jnp.int32),
            pltpu.VMEM((NEG, C), jnp.int32),
            pltpu.VMEM((C, PAIR), jnp.float32),
            pltpu.VMEM((C, PAIR), jnp.float32),
            pltpu.VMEM((NEG, C, PAIR), jnp.float32),
            pltpu.VMEM((C * 8,), jnp.float32),
            pltpu.SemaphoreType.DMA,
        ],
    )(data_flat, emb0p, emb1p)


def _tc_loss_body(ips_ref, data_ref, out_ref):
    ips = ips_ref[...]
    data = data_ref[...]
    pos = ips[:, 0:1]
    negs = ips[:, 1:1 + NEG]
    mask = data[:, 2 + NEG:].astype(jnp.float32)
    pos_l = jnp.sum(-jax.nn.log_sigmoid(jnp.clip(pos, -10.0, 10.0)))
    neg_l = jnp.sum(-jax.nn.log_sigmoid(jnp.clip(-negs, -10.0, 10.0)) * mask)
    out_ref[...] = (pos_l + neg_l).reshape(1, 1)


def _tc_loss(ips, data):
    return pl.pallas_call(
        _tc_loss_body,
        out_shape=jax.ShapeDtypeStruct((1, 1), jnp.float32),
    )(ips, data)


def kernel(data, emb0, emb1):
    emb0p = jnp.concatenate(
        [emb0, jnp.zeros((1, DIM), jnp.float32)], axis=0
    ).reshape((VOCAB + 2) // 2, PAIR)
    emb1p = emb1.reshape(VOCAB // 2, PAIR)
    ips_flat = _sc_ips(data.reshape(-1), emb0p, emb1p)
    ips = ips_flat.reshape(BATCH, 8)
    return _tc_loss(ips, data)[0, 0]


# speedup vs baseline: 1.4418x; 1.4418x over previous
import jax
import jax.numpy as jnp
from jax import lax
from jax.experimental import pallas as pl
from jax.experimental.pallas import tpu as pltpu
from jax.experimental.pallas import tpu_sc as plsc

VOCAB = 1000000
DIM = 64
NEG = 5
BATCH = 16384

NC, NS, L = 2, 16, 16
NW = NC * NS
B_PER_W = BATCH // NW
C = 128
N_CHUNKS = B_PER_W // C
G = C // L
PAIR = 2 * DIM


def _sc_body(dataf_hbm, emb0p_hbm, emb1p_hbm, out_hbm,
             data_v, w_idx, c_idx, n_idx, w_par, c_par, n_par,
             w_rows, c_rows, n_rows, out_v, sem):
    wid = lax.axis_index("c") * NS + lax.axis_index("s")
    base = wid * B_PER_W
    iota = lax.iota(jnp.int32, L)

    def chunk_body(ch, _):
        cbase = base + ch * C
        pltpu.sync_copy(dataf_hbm.at[pl.ds(cbase * 12, C * 12)],
                        data_v.at[pl.ds(0, C * 12)])

        for g in range(G):
            sl = pl.ds(g * L, L)
            rows12 = (g * L + iota) * 12
            vw = plsc.load_gather(data_v, [rows12])
            vc = plsc.load_gather(data_v, [rows12 + 1])
            gw = (vw >= H).astype(jnp.int32)
            gc = (vc >= H).astype(jnp.int32)
            w_idx[sl] = vw - gw * H
            w_par[sl] = gw * DIM
            c_idx[sl] = vc - gc * H
            c_par[sl] = gc * DIM
            for k in range(NEG):
                vn = plsc.load_gather(data_v, [rows12 + (2 + k)])
                gn = (vn >= H).astype(jnp.int32)
                n_idx[k, sl] = vn - gn * H
                n_par[k, sl] = gn * DIM

        cps = [pltpu.async_copy(emb0p_hbm.at[w_idx], w_rows, sem),
               pltpu.async_copy(emb1p_hbm.at[c_idx], c_rows, sem)]
        for k in range(NEG):
            cps.append(pltpu.async_copy(emb1p_hbm.at[n_idx.at[k]],
                                        n_rows.at[k], sem))
        for cp in cps:
            cp.wait()

        def group_body(g, _):
            sl = pl.ds(g * L, L)
            rows = g * L + iota
            wcol = w_par[sl]
            ccol = c_par[sl]
            ncol = [n_par[k, sl] for k in range(NEG)]
            accp = jnp.zeros((L,), jnp.float32)
            accn = [jnp.zeros((L,), jnp.float32) for _ in range(NEG)]
            for d in range(DIM):
                wv = plsc.load_gather(w_rows, [rows, wcol + d])
                cv = plsc.load_gather(c_rows, [rows, ccol + d])
                accp = accp + wv * cv
                for k in range(NEG):
                    nv = plsc.load_gather(
                        n_rows, [jnp.full((L,), k, jnp.int32), rows,
                                 ncol[k] + d])
                    accn[k] = accn[k] + wv * nv
            obase = rows * 8
            plsc.store_scatter(out_v, [obase], accp)
            for k in range(NEG):
                plsc.store_scatter(out_v, [obase + (k + 1)], accn[k])
            return 0

        lax.fori_loop(0, G, group_body, 0)
        pltpu.sync_copy(out_v, out_hbm.at[pl.ds(cbase * 8, C * 8)])
        return 0

    lax.fori_loop(0, N_CHUNKS, chunk_body, 0)


def _sc_ips(data_flat, emb0p, emb1p):
    mesh = plsc.VectorSubcoreMesh(core_axis_name="c", subcore_axis_name="s")
    return pl.kernel(
        _sc_body,
        out_type=jax.ShapeDtypeStruct((BATCH * 8,), jnp.float32),
        mesh=mesh,
        compiler_params=pltpu.CompilerParams(needs_layout_passes=False),
        scratch_types=[
            pltpu.VMEM((C * 12 + L,), jnp.int32),
            pltpu.VMEM((C,), jnp.int32),
            pltpu.VMEM((C,), jnp.int32),
            pltpu.VMEM((NEG, C), jnp.int32),
            pltpu.VMEM((C,), jnp.int32),
            pltpu.VMEM((C,), jnp.int32),
            pltpu.VMEM((NEG, C), jnp.int32),
            pltpu.VMEM((C, PAIR), jnp.float32),
            pltpu.VMEM((C, PAIR), jnp.float32),
            pltpu.VMEM((NEG, C, PAIR), jnp.float32),
            pltpu.VMEM((C * 8,), jnp.float32),
            pltpu.SemaphoreType.DMA,
        ],
    )(data_flat, emb0p, emb1p)


_PACK_LANES = 1024
H = 501760
N_PACK_BLOCKS = H // _PACK_LANES


def _tc_pack_body(lo_ref, hi_ref, out_ref):
    out_ref[...] = jnp.concatenate(
        [lo_ref[...].T, hi_ref[...].T], axis=1)


def _tc_pack(embt):
    last_block = (embt.shape[1] - 1) // _PACK_LANES

    def _hi_map(i):
        return (0, jnp.minimum(N_PACK_BLOCKS + i, last_block))

    return pl.pallas_call(
        _tc_pack_body,
        grid=(N_PACK_BLOCKS,),
        in_specs=[
            pl.BlockSpec((DIM, _PACK_LANES), lambda i: (0, i)),
            pl.BlockSpec((DIM, _PACK_LANES), _hi_map),
        ],
        out_specs=pl.BlockSpec((_PACK_LANES, 2 * DIM), lambda i: (i, 0)),
        out_shape=jax.ShapeDtypeStruct((H, 2 * DIM), jnp.float32),
    )(embt, embt)


def _tc_loss_body(ips_ref, data_ref, out_ref):
    ips = ips_ref[...]
    data = data_ref[...]
    pos = ips[:, 0:1]
    negs = ips[:, 1:1 + NEG]
    mask = data[:, 2 + NEG:].astype(jnp.float32)
    pos_l = jnp.sum(-jax.nn.log_sigmoid(jnp.clip(pos, -10.0, 10.0)))
    neg_l = jnp.sum(-jax.nn.log_sigmoid(jnp.clip(-negs, -10.0, 10.0)) * mask)
    out_ref[...] = (pos_l + neg_l).reshape(1, 1)


def _tc_loss(ips, data):
    return pl.pallas_call(
        _tc_loss_body,
        out_shape=jax.ShapeDtypeStruct((1, 1), jnp.float32),
    )(ips, data)


def kernel(data, emb0, emb1):
    emb0p = _tc_pack(emb0.T)
    emb1p = _tc_pack(emb1.T)
    ips_flat = _sc_ips(data.reshape(-1), emb0p, emb1p)
    ips = ips_flat.reshape(BATCH, 8)
    return _tc_loss(ips, data)[0, 0]


# speedup vs baseline: 2.2634x; 1.5698x over previous
import jax
import jax.numpy as jnp
from jax import lax
from jax.experimental import pallas as pl
from jax.experimental.pallas import tpu as pltpu
from jax.experimental.pallas import tpu_sc as plsc

VOCAB = 1000000
DIM = 64
NEG = 5
BATCH = 16384

NC, NS, L = 2, 16, 16
NW = NC * NS
B_PER_W = BATCH // NW
C = 128
N_CHUNKS = B_PER_W // C
G = C // L
PAIR = 2 * DIM
OC = 16

_PACK_LANES = 4096
N_PACK_BLOCKS = 123
H = _PACK_LANES * N_PACK_BLOCKS


def _tc_pack_body(lo_ref, hi_ref, out_ref):
    out_ref[...] = jnp.concatenate(
        [lo_ref[...].T, hi_ref[...].T], axis=1)


def _tc_pack(embt):
    last_block = (embt.shape[1] - 1) // _PACK_LANES

    def _hi_map(i):
        return (0, jnp.minimum(N_PACK_BLOCKS + i, last_block))

    return pl.pallas_call(
        _tc_pack_body,
        grid=(N_PACK_BLOCKS,),
        in_specs=[
            pl.BlockSpec((DIM, _PACK_LANES), lambda i: (0, i)),
            pl.BlockSpec((DIM, _PACK_LANES), _hi_map),
        ],
        out_specs=pl.BlockSpec((_PACK_LANES, PAIR), lambda i: (i, 0)),
        out_shape=jax.ShapeDtypeStruct((H, PAIR), jnp.float32),
    )(embt, embt)


def _sc_body(dataf_hbm, emb0p_hbm, emb1p_hbm, out_hbm,
             data_v, w_idx, c_idx, n_idx, w_par, c_par, n_par,
             w_rows, c_rows, n_rows, out_v, sem):
    wid = lax.axis_index("c") * NS + lax.axis_index("s")
    base = wid * B_PER_W
    iota = lax.iota(jnp.int32, L)

    def chunk_body(ch, _):
        cbase = base + ch * C
        pltpu.sync_copy(dataf_hbm.at[pl.ds(cbase * 12, C * 12)],
                        data_v.at[pl.ds(0, C * 12)])

        for g in range(G):
            sl = pl.ds(g * L, L)
            rows12 = (g * L + iota) * 12
            obase = (g * L + iota) * OC
            vw = plsc.load_gather(data_v, [rows12])
            vc = plsc.load_gather(data_v, [rows12 + 1])
            gw = (vw >= H).astype(jnp.int32)
            gc = (vc >= H).astype(jnp.int32)
            w_idx[sl] = vw - gw * H
            w_par[sl] = gw * DIM
            c_idx[sl] = vc - gc * H
            c_par[sl] = gc * DIM
            for k in range(NEG):
                vn = plsc.load_gather(data_v, [rows12 + (2 + k)])
                gn = (vn >= H).astype(jnp.int32)
                n_idx[k, sl] = vn - gn * H
                n_par[k, sl] = gn * DIM
            for k in range(NEG):
                vm = plsc.load_gather(data_v, [rows12 + (2 + NEG + k)])
                plsc.store_scatter(out_v, [obase + (8 + k)],
                                   vm.astype(jnp.float32))

        cps = [pltpu.async_copy(emb0p_hbm.at[w_idx], w_rows, sem),
               pltpu.async_copy(emb1p_hbm.at[c_idx], c_rows, sem)]
        for k in range(NEG):
            cps.append(pltpu.async_copy(emb1p_hbm.at[n_idx.at[k]],
                                        n_rows.at[k], sem))
        for cp in cps:
            cp.wait()

        def group_body(g, _):
            sl = pl.ds(g * L, L)
            rows = g * L + iota
            wcol = w_par[sl]
            ccol = c_par[sl]
            ncol = [n_par[k, sl] for k in range(NEG)]
            accp = jnp.zeros((L,), jnp.float32)
            accn = [jnp.zeros((L,), jnp.float32) for _ in range(NEG)]
            for d in range(DIM):
                wv = plsc.load_gather(w_rows, [rows, wcol + d])
                cv = plsc.load_gather(c_rows, [rows, ccol + d])
                accp = accp + wv * cv
                for k in range(NEG):
                    nv = plsc.load_gather(
                        n_rows, [jnp.full((L,), k, jnp.int32), rows,
                                 ncol[k] + d])
                    accn[k] = accn[k] + wv * nv
            obase = rows * OC
            plsc.store_scatter(out_v, [obase], accp)
            for k in range(NEG):
                plsc.store_scatter(out_v, [obase + (k + 1)], accn[k])
            return 0

        lax.fori_loop(0, G, group_body, 0)
        pltpu.sync_copy(out_v, out_hbm.at[pl.ds(cbase * OC, C * OC)])
        return 0

    lax.fori_loop(0, N_CHUNKS, chunk_body, 0)


def _sc_ips(data_flat, emb0p, emb1p):
    mesh = plsc.VectorSubcoreMesh(core_axis_name="c", subcore_axis_name="s")
    return pl.kernel(
        _sc_body,
        out_type=jax.ShapeDtypeStruct((BATCH * OC,), jnp.float32),
        mesh=mesh,
        compiler_params=pltpu.CompilerParams(needs_layout_passes=False),
        scratch_types=[
            pltpu.VMEM((C * 12 + L,), jnp.int32),
            pltpu.VMEM((C,), jnp.int32),
            pltpu.VMEM((C,), jnp.int32),
            pltpu.VMEM((NEG, C), jnp.int32),
            pltpu.VMEM((C,), jnp.int32),
            pltpu.VMEM((C,), jnp.int32),
            pltpu.VMEM((NEG, C), jnp.int32),
            pltpu.VMEM((C, PAIR), jnp.float32),
            pltpu.VMEM((C, PAIR), jnp.float32),
            pltpu.VMEM((NEG, C, PAIR), jnp.float32),
            pltpu.VMEM((C * OC,), jnp.float32),
            pltpu.SemaphoreType.DMA,
        ],
    )(data_flat, emb0p, emb1p)


def _tc_loss_body(x_ref, out_ref):
    x = x_ref[...]
    c = lax.broadcasted_iota(jnp.int32, x.shape, 1) & (OC - 1)
    is_pos = c == 0
    is_neg = (c >= 1) & (c <= NEG)
    w = jnp.roll(x, -7, axis=1)
    t_pos = -jax.nn.log_sigmoid(jnp.clip(x, -10.0, 10.0))
    t_neg = -jax.nn.log_sigmoid(jnp.clip(-x, -10.0, 10.0)) * w
    contrib = jnp.where(is_pos, t_pos, jnp.where(is_neg, t_neg, 0.0))
    out_ref[...] = jnp.sum(contrib).reshape(1, 1)


def _tc_loss(ips_view):
    return pl.pallas_call(
        _tc_loss_body,
        out_shape=jax.ShapeDtypeStruct((1, 1), jnp.float32),
    )(ips_view)


def kernel(data, emb0, emb1):
    emb0p = _tc_pack(emb0.T)
    emb1p = _tc_pack(emb1.T)
    out_flat = _sc_ips(data.reshape(-1), emb0p, emb1p)
    return _tc_loss(out_flat.reshape(BATCH * OC // 128, 128))[0, 0]


# speedup vs baseline: 2.4925x; 1.1013x over previous
import jax
import jax.numpy as jnp
from jax import lax
from jax.experimental import pallas as pl
from jax.experimental.pallas import tpu as pltpu
from jax.experimental.pallas import tpu_sc as plsc

VOCAB = 1000000
DIM = 64
NEG = 5
BATCH = 16384

NC, NS, L = 2, 16, 16
NW = NC * NS
B_PER_W = BATCH // NW
C = 128
N_CHUNKS = B_PER_W // C
G = C // L
PAIR = 2 * DIM
OC = 16

_PACK_LANES = 8192
N_PACK_BLOCKS = 62
H = _PACK_LANES * N_PACK_BLOCKS


def _tc_pack_body(lo_ref, hi_ref, out_ref):
    out_ref[:, 0:DIM] = lo_ref[...].T
    out_ref[:, DIM:PAIR] = hi_ref[...].T


def _tc_pack(embt):
    last_block = (embt.shape[1] - 1) // _PACK_LANES

    def _hi_map(i):
        return (0, jnp.minimum(N_PACK_BLOCKS + i, last_block))

    return pl.pallas_call(
        _tc_pack_body,
        grid=(N_PACK_BLOCKS,),
        in_specs=[
            pl.BlockSpec((DIM, _PACK_LANES), lambda i: (0, i)),
            pl.BlockSpec((DIM, _PACK_LANES), _hi_map),
        ],
        out_specs=pl.BlockSpec((_PACK_LANES, PAIR), lambda i: (i, 0)),
        out_shape=jax.ShapeDtypeStruct((H, PAIR), jnp.float32),
    )(embt, embt)


def _sc_body(dataf_hbm, emb0p_hbm, emb1p_hbm, out_hbm,
             data_v, w_idx, c_idx, n_idx, w_par, c_par, n_par,
             w_rows, c_rows, n_rows, out_v, sem):
    wid = lax.axis_index("c") * NS + lax.axis_index("s")
    base = wid * B_PER_W
    iota = lax.iota(jnp.int32, L)

    def chunk_body(ch, _):
        cbase = base + ch * C
        pltpu.sync_copy(dataf_hbm.at[pl.ds(cbase * 12, C * 12)],
                        data_v.at[pl.ds(0, C * 12)])

        for g in range(G):
            sl = pl.ds(g * L, L)
            rows12 = (g * L + iota) * 12
            obase = (g * L + iota) * OC
            vw = plsc.load_gather(data_v, [rows12])
            vc = plsc.load_gather(data_v, [rows12 + 1])
            gw = (vw >= H).astype(jnp.int32)
            gc = (vc >= H).astype(jnp.int32)
            w_idx[sl] = vw - gw * H
            w_par[sl] = gw * DIM
            c_idx[sl] = vc - gc * H
            c_par[sl] = gc * DIM
            for k in range(NEG):
                vn = plsc.load_gather(data_v, [rows12 + (2 + k)])
                gn = (vn >= H).astype(jnp.int32)
                n_idx[k, sl] = vn - gn * H
                n_par[k, sl] = gn * DIM
            for k in range(NEG):
                vm = plsc.load_gather(data_v, [rows12 + (2 + NEG + k)])
                plsc.store_scatter(out_v, [obase + (8 + k)],
                                   vm.astype(jnp.float32))

        cps = [pltpu.async_copy(emb0p_hbm.at[w_idx], w_rows, sem),
               pltpu.async_copy(emb1p_hbm.at[c_idx], c_rows, sem)]
        for k in range(NEG):
            cps.append(pltpu.async_copy(emb1p_hbm.at[n_idx.at[k]],
                                        n_rows.at[k], sem))
        for cp in cps:
            cp.wait()

        def group_body(g, _):
            sl = pl.ds(g * L, L)
            rows = g * L + iota
            wcol = w_par[sl]
            ccol = c_par[sl]
            ncol = [n_par[k, sl] for k in range(NEG)]
            accp = jnp.zeros((L,), jnp.float32)
            accn = [jnp.zeros((L,), jnp.float32) for _ in range(NEG)]
            for d in range(DIM):
                wv = plsc.load_gather(w_rows, [rows, wcol + d])
                cv = plsc.load_gather(c_rows, [rows, ccol + d])
                accp = accp + wv * cv
                for k in range(NEG):
                    nv = plsc.load_gather(
                        n_rows, [jnp.full((L,), k, jnp.int32), rows,
                                 ncol[k] + d])
                    accn[k] = accn[k] + wv * nv
            obase = rows * OC
            plsc.store_scatter(out_v, [obase], accp)
            for k in range(NEG):
                plsc.store_scatter(out_v, [obase + (k + 1)], accn[k])
            return 0

        lax.fori_loop(0, G, group_body, 0)
        pltpu.sync_copy(out_v, out_hbm.at[pl.ds(cbase * OC, C * OC)])
        return 0

    lax.fori_loop(0, N_CHUNKS, chunk_body, 0)


def _sc_ips(data_flat, emb0p, emb1p):
    mesh = plsc.VectorSubcoreMesh(core_axis_name="c", subcore_axis_name="s")
    return pl.kernel(
        _sc_body,
        out_type=jax.ShapeDtypeStruct((BATCH * OC,), jnp.float32),
        mesh=mesh,
        compiler_params=pltpu.CompilerParams(needs_layout_passes=False),
        scratch_types=[
            pltpu.VMEM((C * 12 + L,), jnp.int32),
            pltpu.VMEM((C,), jnp.int32),
            pltpu.VMEM((C,), jnp.int32),
            pltpu.VMEM((NEG, C), jnp.int32),
            pltpu.VMEM((C,), jnp.int32),
            pltpu.VMEM((C,), jnp.int32),
            pltpu.VMEM((NEG, C), jnp.int32),
            pltpu.VMEM((C, PAIR), jnp.float32),
            pltpu.VMEM((C, PAIR), jnp.float32),
            pltpu.VMEM((NEG, C, PAIR), jnp.float32),
            pltpu.VMEM((C * OC,), jnp.float32),
            pltpu.SemaphoreType.DMA,
        ],
    )(data_flat, emb0p, emb1p)


def _tc_loss_body(x_ref, out_ref):
    x = x_ref[...]
    c = lax.broadcasted_iota(jnp.int32, x.shape, 1) & (OC - 1)
    is_pos = c == 0
    is_neg = (c >= 1) & (c <= NEG)
    w = jnp.roll(x, -7, axis=1)
    t_pos = -jax.nn.log_sigmoid(jnp.clip(x, -10.0, 10.0))
    t_neg = -jax.nn.log_sigmoid(jnp.clip(-x, -10.0, 10.0)) * w
    contrib = jnp.where(is_pos, t_pos, jnp.where(is_neg, t_neg, 0.0))
    out_ref[...] = jnp.sum(contrib).reshape(1, 1)


def _tc_loss(ips_view):
    return pl.pallas_call(
        _tc_loss_body,
        out_shape=jax.ShapeDtypeStruct((1, 1), jnp.float32),
    )(ips_view)


def kernel(data, emb0, emb1):
    emb0p = _tc_pack(emb0.T)
    emb1p = _tc_pack(emb1.T)
    out_flat = _sc_ips(data.reshape(-1), emb0p, emb1p)
    return _tc_loss(out_flat.reshape(BATCH * OC // 128, 128))[0, 0]


# speedup vs baseline: 2.5423x; 1.0200x over previous
import jax
import jax.numpy as jnp
from jax import lax
from jax.experimental import pallas as pl
from jax.experimental.pallas import tpu as pltpu
from jax.experimental.pallas import tpu_sc as plsc

VOCAB = 1000000
DIM = 64
NEG = 5
BATCH = 16384

NC, NS, L = 2, 16, 16
NW = NC * NS
B_PER_W = BATCH // NW
C = 64
N_CHUNKS = B_PER_W // C
G = C // L
PAIR = 2 * DIM
OC = 16

_PACK_LANES = 8192
N_PACK_BLOCKS = 62
H = _PACK_LANES * N_PACK_BLOCKS


def _tc_pack_body(lo_ref, hi_ref, out_ref):
    out_ref[:, 0:DIM] = lo_ref[...].T
    out_ref[:, DIM:PAIR] = hi_ref[...].T


def _tc_pack(embt):
    last_block = (embt.shape[1] - 1) // _PACK_LANES

    def _hi_map(i):
        return (0, jnp.minimum(N_PACK_BLOCKS + i, last_block))

    return pl.pallas_call(
        _tc_pack_body,
        grid=(N_PACK_BLOCKS,),
        in_specs=[
            pl.BlockSpec((DIM, _PACK_LANES), lambda i: (0, i)),
            pl.BlockSpec((DIM, _PACK_LANES), _hi_map),
        ],
        out_specs=pl.BlockSpec((_PACK_LANES, PAIR), lambda i: (i, 0)),
        out_shape=jax.ShapeDtypeStruct((H, PAIR), jnp.float32),
    )(embt, embt)


def _sc_body(dataf_hbm, emb0p_hbm, emb1p_hbm, out_hbm,
             data_v, w_idx, c_idx, n_idx, w_par, c_par, n_par,
             w_rows, c_rows, n_rows, out_v, sem):
    wid = lax.axis_index("c") * NS + lax.axis_index("s")
    base = wid * B_PER_W
    iota = lax.iota(jnp.int32, L)

    DB = C * 12 + L

    def stage_fire(ch, b):
        cbase = base + ch * C
        dv = b * DB
        pltpu.sync_copy(dataf_hbm.at[pl.ds(cbase * 12, C * 12)],
                        data_v.at[pl.ds(dv, C * 12)])
        for g in range(G):
            sl = pl.ds(b * C + g * L, L)
            rows12 = dv + (g * L + iota) * 12
            vw = plsc.load_gather(data_v, [rows12])
            vc = plsc.load_gather(data_v, [rows12 + 1])
            gw = (vw >= H).astype(jnp.int32)
            gc = (vc >= H).astype(jnp.int32)
            w_idx[sl] = vw - gw * H
            w_par[sl] = gw * DIM
            c_idx[sl] = vc - gc * H
            c_par[sl] = gc * DIM
            for k in range(NEG):
                vn = plsc.load_gather(data_v, [rows12 + (2 + k)])
                gn = (vn >= H).astype(jnp.int32)
                n_idx[pl.ds((b * NEG + k) * C + g * L, L)] = vn - gn * H
                n_par[pl.ds((b * NEG + k) * C + g * L, L)] = gn * DIM
        pltpu.async_copy(emb0p_hbm.at[w_idx.at[pl.ds(b * C, C)]],
                         w_rows.at[pl.ds(b * C, C)], sem.at[b])
        pltpu.async_copy(emb1p_hbm.at[c_idx.at[pl.ds(b * C, C)]],
                         c_rows.at[pl.ds(b * C, C)], sem.at[b])
        for k in range(NEG):
            nk = (b * NEG + k) * C
            pltpu.async_copy(emb1p_hbm.at[n_idx.at[pl.ds(nk, C)]],
                             n_rows.at[pl.ds(nk, C)], sem.at[b])

    def drain(b):
        pltpu.make_async_copy(emb0p_hbm.at[pl.ds(0, C)],
                              w_rows.at[pl.ds(b * C, C)], sem.at[b]).wait()
        pltpu.make_async_copy(emb0p_hbm.at[pl.ds(0, C)],
                              c_rows.at[pl.ds(b * C, C)], sem.at[b]).wait()
        for k in range(NEG):
            nk = (b * NEG + k) * C
            pltpu.make_async_copy(emb0p_hbm.at[pl.ds(0, C)],
                                  n_rows.at[pl.ds(nk, C)], sem.at[b]).wait()

    def compute(ch, b):
        cbase = base + ch * C
        dv = b * DB

        def group_body(g, _):
            rows = g * L + iota
            rows12 = dv + rows * 12
            wcol = w_par[pl.ds(b * C + g * L, L)]
            ccol = c_par[pl.ds(b * C + g * L, L)]
            ncol = [n_par[pl.ds((b * NEG + k) * C + g * L, L)]
                    for k in range(NEG)]
            accp = jnp.zeros((L,), jnp.float32)
            accn = [jnp.zeros((L,), jnp.float32) for _ in range(NEG)]
            brows = b * C + rows
            for d in range(DIM):
                wv = plsc.load_gather(w_rows, [brows, wcol + d])
                cv = plsc.load_gather(c_rows, [brows, ccol + d])
                accp = accp + wv * cv
                for k in range(NEG):
                    nv = plsc.load_gather(
                        n_rows, [(b * NEG + k) * C + rows, ncol[k] + d])
                    accn[k] = accn[k] + wv * nv
            obase = rows * OC
            plsc.store_scatter(out_v, [obase], accp)
            for k in range(NEG):
                plsc.store_scatter(out_v, [obase + (k + 1)], accn[k])
            for k in range(NEG):
                vm = plsc.load_gather(data_v, [rows12 + (2 + NEG + k)])
                plsc.store_scatter(out_v, [obase + (8 + k)],
                                   vm.astype(jnp.float32))
            return 0

        lax.fori_loop(0, G, group_body, 0)
        pltpu.sync_copy(out_v, out_hbm.at[pl.ds(cbase * OC, C * OC)])

    stage_fire(0, 0)

    def pair_body(i, _):
        c0 = 2 * i
        stage_fire(c0 + 1, 1)
        drain(0)
        compute(c0, 0)

        @pl.when(c0 + 2 < N_CHUNKS)
        def _prefetch():
            stage_fire(c0 + 2, 0)

        drain(1)
        compute(c0 + 1, 1)
        return 0

    lax.fori_loop(0, N_CHUNKS // 2, pair_body, 0)


def _sc_ips(data_flat, emb0p, emb1p):
    mesh = plsc.VectorSubcoreMesh(core_axis_name="c", subcore_axis_name="s")
    return pl.kernel(
        _sc_body,
        out_type=jax.ShapeDtypeStruct((BATCH * OC,), jnp.float32),
        mesh=mesh,
        compiler_params=pltpu.CompilerParams(needs_layout_passes=False),
        scratch_types=[
            pltpu.VMEM((2 * (C * 12 + L),), jnp.int32),
            pltpu.VMEM((2 * C,), jnp.int32),
            pltpu.VMEM((2 * C,), jnp.int32),
            pltpu.VMEM((2 * NEG * C,), jnp.int32),
            pltpu.VMEM((2 * C,), jnp.int32),
            pltpu.VMEM((2 * C,), jnp.int32),
            pltpu.VMEM((2 * NEG * C,), jnp.int32),
            pltpu.VMEM((2 * C, PAIR), jnp.float32),
            pltpu.VMEM((2 * C, PAIR), jnp.float32),
            pltpu.VMEM((2 * NEG * C, PAIR), jnp.float32),
            pltpu.VMEM((C * OC,), jnp.float32),
            pltpu.SemaphoreType.DMA((2,)),
        ],
    )(data_flat, emb0p, emb1p)


def _tc_loss_body(x_ref, out_ref):
    x = x_ref[...]
    c = lax.broadcasted_iota(jnp.int32, x.shape, 1) & (OC - 1)
    is_pos = c == 0
    is_neg = (c >= 1) & (c <= NEG)
    w = jnp.roll(x, -7, axis=1)
    t_pos = -jax.nn.log_sigmoid(jnp.clip(x, -10.0, 10.0))
    t_neg = -jax.nn.log_sigmoid(jnp.clip(-x, -10.0, 10.0)) * w
    contrib = jnp.where(is_pos, t_pos, jnp.where(is_neg, t_neg, 0.0))
    out_ref[...] = jnp.sum(contrib).reshape(1, 1)


def _tc_loss(ips_view):
    return pl.pallas_call(
        _tc_loss_body,
        out_shape=jax.ShapeDtypeStruct((1, 1), jnp.float32),
    )(ips_view)


def kernel(data, emb0, emb1):
    emb0p = _tc_pack(emb0.T)
    emb1p = _tc_pack(emb1.T)
    out_flat = _sc_ips(data.reshape(-1), emb0p, emb1p)
    return _tc_loss(out_flat.reshape(BATCH * OC // 128, 128))[0, 0]


# speedup vs baseline: 2.5455x; 1.0012x over previous
import jax
import jax.numpy as jnp
from jax import lax
from jax.experimental import pallas as pl
from jax.experimental.pallas import tpu as pltpu
from jax.experimental.pallas import tpu_sc as plsc

VOCAB = 1000000
DIM = 64
NEG = 5
BATCH = 16384

NC, NS, L = 2, 16, 16
NW = NC * NS
B_PER_W = BATCH // NW
C = 64
N_CHUNKS = B_PER_W // C
G = C // L
PAIR = 2 * DIM
OC = 16

_PACK_LANES = 8192
N_PACK_BLOCKS = 62
H = _PACK_LANES * N_PACK_BLOCKS


def _tc_pack_body(lo_ref, hi_ref, out_ref):
    out_ref[:, 0:DIM] = lo_ref[...].T
    out_ref[:, DIM:PAIR] = hi_ref[...].T


def _tc_pack(embt):
    last_block = (embt.shape[1] - 1) // _PACK_LANES

    def _hi_map(i):
        return (0, jnp.minimum(N_PACK_BLOCKS + i, last_block))

    return pl.pallas_call(
        _tc_pack_body,
        grid=(N_PACK_BLOCKS,),
        in_specs=[
            pl.BlockSpec((DIM, _PACK_LANES), lambda i: (0, i)),
            pl.BlockSpec((DIM, _PACK_LANES), _hi_map),
        ],
        out_specs=pl.BlockSpec((_PACK_LANES, PAIR), lambda i: (i, 0)),
        out_shape=jax.ShapeDtypeStruct((H, PAIR), jnp.float32),
    )(embt, embt)


def _sc_body(dataf_hbm, emb0p_hbm, emb1p_hbm, out_hbm,
             data_v, w_idx, e1_idx, w_par, e1_par,
             w_rows, e1_rows, out_v, sem):
    wid = lax.axis_index("c") * NS + lax.axis_index("s")
    base = wid * B_PER_W
    iota = lax.iota(jnp.int32, L)

    DB = C * 12 + L
    S = NEG + 1

    def stage_fire(ch, b):
        cbase = base + ch * C
        dv = b * DB
        pltpu.sync_copy(dataf_hbm.at[pl.ds(cbase * 12, C * 12)],
                        data_v.at[pl.ds(dv, C * 12)])
        for g in range(G):
            rows12 = dv + (g * L + iota) * 12
            vw = plsc.load_gather(data_v, [rows12])
            gw = (vw >= H).astype(jnp.int32)
            w_idx[pl.ds(b * C + g * L, L)] = vw - gw * H
            w_par[pl.ds(b * C + g * L, L)] = gw * DIM
            for j in range(S):
                vx = plsc.load_gather(data_v, [rows12 + (1 + j)])
                gx = (vx >= H).astype(jnp.int32)
                e1_idx[pl.ds((b * S + j) * C + g * L, L)] = vx - gx * H
                e1_par[pl.ds((b * S + j) * C + g * L, L)] = gx * DIM
        pltpu.async_copy(emb0p_hbm.at[w_idx.at[pl.ds(b * C, C)]],
                         w_rows.at[pl.ds(b * C, C)], sem.at[b])
        for m in range(S // 2):
            ofs = (b * S + 2 * m) * C
            pltpu.async_copy(emb1p_hbm.at[e1_idx.at[pl.ds(ofs, 2 * C)]],
                             e1_rows.at[pl.ds(ofs, 2 * C)], sem.at[b])

    def drain(b):
        pltpu.make_async_copy(emb0p_hbm.at[pl.ds(0, C)],
                              w_rows.at[pl.ds(b * C, C)], sem.at[b]).wait()
        for m in range(S // 2):
            ofs = (b * S + 2 * m) * C
            pltpu.make_async_copy(emb0p_hbm.at[pl.ds(0, 2 * C)],
                                  e1_rows.at[pl.ds(ofs, 2 * C)],
                                  sem.at[b]).wait()

    def compute(ch, b):
        cbase = base + ch * C
        dv = b * DB

        def group_body(g, _):
            rows = g * L + iota
            rows12 = dv + rows * 12
            wcol = w_par[pl.ds(b * C + g * L, L)]
            xcol = [e1_par[pl.ds((b * S + j) * C + g * L, L)]
                    for j in range(S)]
            accs = [jnp.zeros((L,), jnp.float32) for _ in range(S)]
            brows = b * C + rows
            for d in range(DIM):
                wv = plsc.load_gather(w_rows, [brows, wcol + d])
                for j in range(S):
                    xv = plsc.load_gather(
                        e1_rows, [(b * S + j) * C + rows, xcol[j] + d])
                    accs[j] = accs[j] + wv * xv
            obase = rows * OC
            for j in range(S):
                plsc.store_scatter(out_v, [obase + j], accs[j])
            for k in range(NEG):
                vm = plsc.load_gather(data_v, [rows12 + (2 + NEG + k)])
                plsc.store_scatter(out_v, [obase + (8 + k)],
                                   vm.astype(jnp.float32))
            return 0

        lax.fori_loop(0, G, group_body, 0)
        pltpu.sync_copy(out_v, out_hbm.at[pl.ds(cbase * OC, C * OC)])

    stage_fire(0, 0)

    def pair_body(i, _):
        c0 = 2 * i
        stage_fire(c0 + 1, 1)
        drain(0)
        compute(c0, 0)

        @pl.when(c0 + 2 < N_CHUNKS)
        def _prefetch():
            stage_fire(c0 + 2, 0)

        drain(1)
        compute(c0 + 1, 1)
        return 0

    lax.fori_loop(0, N_CHUNKS // 2, pair_body, 0)


def _sc_ips(data_flat, emb0p, emb1p):
    mesh = plsc.VectorSubcoreMesh(core_axis_name="c", subcore_axis_name="s")
    return pl.kernel(
        _sc_body,
        out_type=jax.ShapeDtypeStruct((BATCH * OC,), jnp.float32),
        mesh=mesh,
        compiler_params=pltpu.CompilerParams(needs_layout_passes=False),
        scratch_types=[
            pltpu.VMEM((2 * (C * 12 + L),), jnp.int32),
            pltpu.VMEM((2 * C,), jnp.int32),
            pltpu.VMEM((2 * (NEG + 1) * C,), jnp.int32),
            pltpu.VMEM((2 * C,), jnp.int32),
            pltpu.VMEM((2 * (NEG + 1) * C,), jnp.int32),
            pltpu.VMEM((2 * C, PAIR), jnp.float32),
            pltpu.VMEM((2 * (NEG + 1) * C, PAIR), jnp.float32),
            pltpu.VMEM((C * OC,), jnp.float32),
            pltpu.SemaphoreType.DMA((2,)),
        ],
    )(data_flat, emb0p, emb1p)


def _tc_loss_body(x_ref, out_ref):
    x = x_ref[...]
    c = lax.broadcasted_iota(jnp.int32, x.shape, 1) & (OC - 1)
    is_pos = c == 0
    is_neg = (c >= 1) & (c <= NEG)
    w = jnp.roll(x, -7, axis=1)
    t_pos = -jax.nn.log_sigmoid(jnp.clip(x, -10.0, 10.0))
    t_neg = -jax.nn.log_sigmoid(jnp.clip(-x, -10.0, 10.0)) * w
    contrib = jnp.where(is_pos, t_pos, jnp.where(is_neg, t_neg, 0.0))
    out_ref[...] = jnp.sum(contrib).reshape(1, 1)


def _tc_loss(ips_view):
    return pl.pallas_call(
        _tc_loss_body,
        out_shape=jax.ShapeDtypeStruct((1, 1), jnp.float32),
    )(ips_view)


def kernel(data, emb0, emb1):
    emb0p = _tc_pack(emb0.T)
    emb1p = _tc_pack(emb1.T)
    out_flat = _sc_ips(data.reshape(-1), emb0p, emb1p)
    return _tc_loss(out_flat.reshape(BATCH * OC // 128, 128))[0, 0]


# speedup vs baseline: 3.0492x; 1.1979x over previous
import jax
import jax.numpy as jnp
from jax import lax
from jax.experimental import pallas as pl
from jax.experimental.pallas import tpu as pltpu
from jax.experimental.pallas import tpu_sc as plsc

VOCAB = 1000000
DIM = 64
NEG = 5
BATCH = 16384

NC, NS, L = 2, 16, 16
NW = NC * NS
B_PER_W = BATCH // NW
C = 64
N_CHUNKS = B_PER_W // C
G = C // L
PAIR = 2 * DIM
OC = 16

_PACK_LANES = 8192
N_PACK_BLOCKS = 62
H = _PACK_LANES * N_PACK_BLOCKS


def _tc_pack_body(lo_ref, hi_ref, out_ref):
    out_ref[:, 0:DIM] = lo_ref[...].T
    out_ref[:, DIM:PAIR] = hi_ref[...].T


def _tc_pack(embt):
    last_block = (embt.shape[1] - 1) // _PACK_LANES

    def _hi_map(i):
        return (0, jnp.minimum(N_PACK_BLOCKS + i, last_block))

    return pl.pallas_call(
        _tc_pack_body,
        grid=(N_PACK_BLOCKS,),
        in_specs=[
            pl.BlockSpec((DIM, _PACK_LANES), lambda i: (0, i)),
            pl.BlockSpec((DIM, _PACK_LANES), _hi_map),
        ],
        out_specs=pl.BlockSpec((_PACK_LANES, PAIR), lambda i: (i, 0)),
        out_shape=jax.ShapeDtypeStruct((H, PAIR), jnp.float32),
    )(embt, embt)


def _sc_body(dataf_hbm, emb0p_hbm, emb1p_hbm, out_hbm,
             data_v, w_idx, e1_idx, w_par, e1_par,
             w_rows, e1_rows, out_v, sem):
    wid = lax.axis_index("c") * NS + lax.axis_index("s")
    base = wid * B_PER_W
    iota = lax.iota(jnp.int32, L)

    DB = C * 12 + L
    S = NEG + 1

    def stage_fire(ch, b):
        cbase = base + ch * C
        dv = b * DB
        pltpu.sync_copy(dataf_hbm.at[pl.ds(cbase * 12, C * 12)],
                        data_v.at[pl.ds(dv, C * 12)])
        for g in range(G):
            rows12 = dv + (g * L + iota) * 12
            vw = plsc.load_gather(data_v, [rows12])
            gw = (vw >= H).astype(jnp.int32)
            w_idx[pl.ds(b * C + g * L, L)] = vw - gw * H
            w_par[pl.ds(b * C + g * L, L)] = gw * DIM
            for j in range(S):
                vx = plsc.load_gather(data_v, [rows12 + (1 + j)])
                gx = (vx >= H).astype(jnp.int32)
                e1_idx[pl.ds((b * S + j) * C + g * L, L)] = vx - gx * H
                e1_par[pl.ds((b * S + j) * C + g * L, L)] = gx * DIM
        pltpu.async_copy(emb0p_hbm.at[w_idx.at[pl.ds(b * C, C)]],
                         w_rows.at[pl.ds(b * C, C)], sem.at[b])
        for m in range(S // 2):
            ofs = (b * S + 2 * m) * C
            pltpu.async_copy(emb1p_hbm.at[e1_idx.at[pl.ds(ofs, 2 * C)]],
                             e1_rows.at[pl.ds(ofs, 2 * C)], sem.at[b])

    def drain(b):
        pltpu.make_async_copy(emb0p_hbm.at[pl.ds(0, C)],
                              w_rows.at[pl.ds(b * C, C)], sem.at[b]).wait()
        for m in range(S // 2):
            ofs = (b * S + 2 * m) * C
            pltpu.make_async_copy(emb0p_hbm.at[pl.ds(0, 2 * C)],
                                  e1_rows.at[pl.ds(ofs, 2 * C)],
                                  sem.at[b]).wait()

    def compute(ch, b):
        cbase = base + ch * C
        dv = b * DB

        def group_body(g, _):
            rows = g * L + iota
            rows12 = dv + rows * 12
            wcol = w_par[pl.ds(b * C + g * L, L)]
            xcol = [e1_par[pl.ds((b * S + j) * C + g * L, L)]
                    for j in range(S)]
            accs = [jnp.zeros((L,), jnp.float32) for _ in range(S)]
            brows = b * C + rows
            for d in range(DIM):
                rot = (iota + d) & (DIM - 1)
                wv = plsc.load_gather(w_rows, [brows, wcol + rot])
                for j in range(S):
                    xv = plsc.load_gather(
                        e1_rows, [(b * S + j) * C + rows, xcol[j] + rot])
                    accs[j] = accs[j] + wv * xv
            obase = rows * OC
            for j in range(S):
                plsc.store_scatter(out_v, [obase + j], accs[j])
            for k in range(NEG):
                vm = plsc.load_gather(data_v, [rows12 + (2 + NEG + k)])
                plsc.store_scatter(out_v, [obase + (8 + k)],
                                   vm.astype(jnp.float32))
            return 0

        lax.fori_loop(0, G, group_body, 0)
        pltpu.sync_copy(out_v, out_hbm.at[pl.ds(cbase * OC, C * OC)])

    stage_fire(0, 0)

    def pair_body(i, _):
        c0 = 2 * i
        stage_fire(c0 + 1, 1)
        drain(0)
        compute(c0, 0)

        @pl.when(c0 + 2 < N_CHUNKS)
        def _prefetch():
            stage_fire(c0 + 2, 0)

        drain(1)
        compute(c0 + 1, 1)
        return 0

    lax.fori_loop(0, N_CHUNKS // 2, pair_body, 0)


def _sc_ips(data_flat, emb0p, emb1p):
    mesh = plsc.VectorSubcoreMesh(core_axis_name="c", subcore_axis_name="s")
    return pl.kernel(
        _sc_body,
        out_type=jax.ShapeDtypeStruct((BATCH * OC,), jnp.float32),
        mesh=mesh,
        compiler_params=pltpu.CompilerParams(needs_layout_passes=False),
        scratch_types=[
            pltpu.VMEM((2 * (C * 12 + L),), jnp.int32),
            pltpu.VMEM((2 * C,), jnp.int32),
            pltpu.VMEM((2 * (NEG + 1) * C,), jnp.int32),
            pltpu.VMEM((2 * C,), jnp.int32),
            pltpu.VMEM((2 * (NEG + 1) * C,), jnp.int32),
            pltpu.VMEM((2 * C, PAIR), jnp.float32),
            pltpu.VMEM((2 * (NEG + 1) * C, PAIR), jnp.float32),
            pltpu.VMEM((C * OC,), jnp.float32),
            pltpu.SemaphoreType.DMA((2,)),
        ],
    )(data_flat, emb0p, emb1p)


def _tc_loss_body(x_ref, out_ref):
    x = x_ref[...]
    c = lax.broadcasted_iota(jnp.int32, x.shape, 1) & (OC - 1)
    is_pos = c == 0
    is_neg = (c >= 1) & (c <= NEG)
    w = jnp.roll(x, -7, axis=1)
    t_pos = -jax.nn.log_sigmoid(jnp.clip(x, -10.0, 10.0))
    t_neg = -jax.nn.log_sigmoid(jnp.clip(-x, -10.0, 10.0)) * w
    contrib = jnp.where(is_pos, t_pos, jnp.where(is_neg, t_neg, 0.0))
    out_ref[...] = jnp.sum(contrib).reshape(1, 1)


def _tc_loss(ips_view):
    return pl.pallas_call(
        _tc_loss_body,
        out_shape=jax.ShapeDtypeStruct((1, 1), jnp.float32),
    )(ips_view)


def kernel(data, emb0, emb1):
    emb0p = _tc_pack(emb0.T)
    emb1p = _tc_pack(emb1.T)
    out_flat = _sc_ips(data.reshape(-1), emb0p, emb1p)
    return _tc_loss(out_flat.reshape(BATCH * OC // 128, 128))[0, 0]
